# baseline (speedup 1.0000x reference)
import functools

import jax
import jax.numpy as jnp
from jax import lax
from jax.experimental import pallas as pl
from jax.experimental.pallas import tpu as pltpu
from jax.experimental.pallas import tpu_sc as plsc

NC = 2
NS = 16
NW = NC * NS
BG = 64
BG_LOG2 = BG.bit_length() - 1

_DOT_KW = dict(preferred_element_type=jnp.float32, precision=lax.Precision.HIGHEST)


def _dot_t(a, w):
    return lax.dot_general(a, w, (((1,), (1,)), ((), ())), **_DOT_KW)


def _in_fc_body(x_ref, w_ref, b_ref, o_ref):
    o_ref[...] = _dot_t(x_ref[...], w_ref[...]) + b_ref[...]


def _in_fc(x, W, b, rb):
    n, d = x.shape
    return pl.pallas_call(
        _in_fc_body,
        grid=(n // rb,),
        in_specs=[
            pl.BlockSpec((rb, d), lambda i: (i, 0)),
            pl.BlockSpec((d, d), lambda i: (0, 0)),
            pl.BlockSpec((1, d), lambda i: (0, 0)),
        ],
        out_specs=pl.BlockSpec((rb, d), lambda i: (i, 0)),
        out_shape=jax.ShapeDtypeStruct((n, d), jnp.float32),
    )(x, W, b.reshape(1, d))


def _layer_body(p_ref, h_ref, wrel_ref, brel_ref, wroot_ref, o_ref):
    agg = p_ref[0] + p_ref[1]
    acc = _dot_t(agg, wrel_ref[...]) + brel_ref[...]
    acc = acc + _dot_t(h_ref[...], wroot_ref[...])
    o_ref[...] = jnp.maximum(acc, 0.0)


def _layer(parts, h, Wrel, brel, Wroot, rb):
    n, d = h.shape
    return pl.pallas_call(
        _layer_body,
        grid=(n // rb,),
        in_specs=[
            pl.BlockSpec((2, rb, d), lambda i: (0, i, 0)),
            pl.BlockSpec((rb, d), lambda i: (i, 0)),
            pl.BlockSpec((d, d), lambda i: (0, 0)),
            pl.BlockSpec((1, d), lambda i: (0, 0)),
            pl.BlockSpec((d, d), lambda i: (0, 0)),
        ],
        out_specs=pl.BlockSpec((rb, d), lambda i: (i, 0)),
        out_shape=jax.ShapeDtypeStruct((n, d), jnp.float32),
    )(parts, h, Wrel, brel.reshape(1, d), Wroot)


@functools.lru_cache(maxsize=None)
def _make_segsum(n_pad, d, nb, reuse):
    n_sh = n_pad + 8
    rows_t = n_pad // NS
    nck = 16
    k_b = n_pad // 8
    nbh = nb // 2
    assert nbh % nck == 0 and rows_t % 8 == 0 and k_b % 16 == 0
    mesh = plsc.VectorSubcoreMesh(
        core_axis_name="c", subcore_axis_name="s", num_cores=NC, num_subcores=NS
    )

    agg_t = jax.ShapeDtypeStruct((NC, n_pad, d), jnp.float32)
    lists_t = jax.ShapeDtypeStruct((NW * 2, nbh, BG), jnp.int32)
    out_type = agg_t if reuse else (agg_t, lists_t, lists_t)

    @functools.partial(
        pl.kernel,
        out_type=out_type,
        mesh=mesh,
        compiler_params=pltpu.CompilerParams(needs_layout_passes=False),
        scratch_types=[
            pltpu.VMEM((nck, BG), jnp.int32),
            pltpu.VMEM((nck, BG), jnp.int32),
            pltpu.VMEM((k_b,), jnp.int32),
            pltpu.VMEM((nbh, BG), jnp.int32),
            pltpu.VMEM((nbh, BG), jnp.int32),
            pltpu.VMEM((8, d), jnp.float32),
            pltpu.VMEM((BG, d), jnp.float32),
            pltpu.VMEM((BG, d), jnp.float32),
            pltpu.VMEM_SHARED((n_sh, d), jnp.float32),
            pltpu.SemaphoreType.DMA,
            pltpu.SemaphoreType.DMA,
        ],
    )
    def segsum(*args):
        if reuse:
            (h_hbm, ss_hbm, ds_hbm, out_hbm,
             sv, dv, offs, ss2, ds2, zb, rb0, rb1, acc, sem0, sem1) = args
            src_hbm = dst_hbm = None
        else:
            (h_hbm, src_hbm, dst_hbm, out_hbm, ss_hbm, ds_hbm,
             sv, dv, offs, ss2, ds2, zb, rb0, rb1, acc, sem0, sem1) = args
        c = lax.axis_index("c")
        s = lax.axis_index("s")
        wid = s * NC + c

        def zrow(r, carry):
            for cg in range(d // 16):
                zb[r, pl.ds(cg * 16, 16)] = jnp.zeros((16,), jnp.float32)
            return carry
        lax.fori_loop(0, 8, zrow, 0)
        base = s * rows_t

        def zcopy(k, carry):
            off = pl.multiple_of(base + k * 8, 8)
            pltpu.sync_copy(zb, acc.at[pl.ds(off, 8)])
            return carry
        lax.fori_loop(0, rows_t // 8, zcopy, 0)
        plsc.subcore_barrier()

        for ph in range(2):
            hbase = ph * nbh

            if reuse:
                pltpu.async_copy(ss_hbm.at[wid * 2 + ph], ss2, sem0)
                pltpu.async_copy(ds_hbm.at[wid * 2 + ph], ds2, sem1)
                pltpu.make_async_copy(ss_hbm.at[wid * 2 + ph], ss2, sem0).wait()
                pltpu.make_async_copy(ds_hbm.at[wid * 2 + ph], ds2, sem1).wait()
                _sort = False
            else:
                _sort = True

            def hzero(k, carry):
                off = pl.multiple_of(k * 16, 16)
                offs[pl.ds(off, 16)] = jnp.zeros((16,), jnp.int32)
                return carry
            if _sort:
                lax.fori_loop(0, k_b // 16, hzero, 0)

            def hist_chunk(ch, carry):
                off = pl.multiple_of(hbase + ch * nck, nck)
                pltpu.sync_copy(src_hbm.at[wid, pl.ds(off, nck)], sv)
                for r in range(nck):
                    for cg in range(BG // 16):
                        b = sv[r, pl.ds(cg * 16, 16)] >> 3
                        g = plsc.load_gather(offs, [b])
                        cnt, last = plsc.scan_count(b)
                        plsc.store_scatter(offs, [b], g + cnt, mask=last)
                return carry
            if _sort:
                lax.fori_loop(0, nbh // nck, hist_chunk, 0)

            def pfx(k, carry):
                off = pl.multiple_of(k * 16, 16)
                v = offs[pl.ds(off, 16)]
                total = jnp.sum(v)
                offs[pl.ds(off, 16)] = carry + plsc.cumsum(v) - v
                return carry + total
            if _sort:
                lax.fori_loop(0, k_b // 16, pfx, jnp.int32(0))

            def perm_chunk(ch, carry):
                off = pl.multiple_of(hbase + ch * nck, nck)
                pltpu.sync_copy(src_hbm.at[wid, pl.ds(off, nck)], sv)
                pltpu.sync_copy(dst_hbm.at[wid, pl.ds(off, nck)], dv)
                for r in range(nck):
                    for cg in range(BG // 16):
                        sl = pl.ds(cg * 16, 16)
                        s_v = sv[r, sl]
                        d_v = dv[r, sl]
                        b = s_v >> 3
                        g = plsc.load_gather(offs, [b])
                        cnt, last = plsc.scan_count(b)
                        p = g + cnt - 1
                        row = p >> BG_LOG2
                        col = p & (BG - 1)
                        plsc.store_scatter(ss2, [row, col], s_v)
                        plsc.store_scatter(ds2, [row, col], d_v)
                        plsc.store_scatter(offs, [b], p + 1, mask=last)
                return carry
            if _sort:
                lax.fori_loop(0, nbh // nck, perm_chunk, 0)
                pltpu.sync_copy(ss2, ss_hbm.at[wid * 2 + ph])
                pltpu.sync_copy(ds2, ds_hbm.at[wid * 2 + ph])

            pltpu.async_copy(h_hbm.at[ss2.at[0]], rb0, sem0)

            def body(jj, carry2):
                j0 = jj * 2
                j1 = j0 + 1
                pltpu.make_async_copy(h_hbm.at[ss2.at[j0]], rb0, sem0).wait()
                pltpu.async_copy(h_hbm.at[ss2.at[j1]], rb1, sem1)
                pltpu.sync_copy(rb0, acc.at[ds2.at[j0]], add=True)
                pltpu.make_async_copy(h_hbm.at[ss2.at[j1]], rb1, sem1).wait()

                @pl.when(j0 + 2 < nbh)
                def _():
                    pltpu.async_copy(h_hbm.at[ss2.at[j0 + 2]], rb0, sem0)

                pltpu.sync_copy(rb1, acc.at[ds2.at[j1]], add=True)
                return carry2

            lax.fori_loop(0, nbh // 2, body, 0)

        plsc.subcore_barrier()

        pltpu.sync_copy(acc.at[pl.ds(s * rows_t, rows_t)],
                        out_hbm.at[c, pl.ds(s * rows_t, rows_t)])

    return segsum


def kernel(x, edge_index, W_in, b_in, W_rel1, b_rel1, W_root1,
           W_rel2, b_rel2, W_root2):
    n, d = x.shape
    e = edge_index.shape[1]

    per_w = NW * BG
    nb = -(-(-(-e // per_w)) // 32) * 32
    e_pad = nb * per_w

    n_pad = -(-n // (8 * NS)) * 8 * NS

    src = jnp.pad(edge_index[0], (0, e_pad - e))
    dst = jnp.pad(edge_index[1], (0, e_pad - e), constant_values=n_pad)
    src3 = src.reshape(NW, nb, BG)
    dst3 = dst.reshape(NW, nb, BG)

    segsum_sort = _make_segsum(n_pad, d, nb, False)
    segsum_reuse = _make_segsum(n_pad, d, nb, True)
    rb = 1000 if n % 1000 == 0 else 8

    h = _in_fc(x, W_in, b_in, rb)
    parts, ss_lists, ds_lists = segsum_sort(h, src3, dst3)
    h1 = _layer(parts, h, W_rel1, b_rel1, W_root1, rb)
    parts2 = segsum_reuse(h1, ss_lists, ds_lists)
    h2 = _layer(parts2, h1, W_rel2, b_rel2, W_root2, rb)
    return h2


# speedup vs baseline: 1.2213x; 1.2213x over previous
import functools

import jax
import jax.numpy as jnp
from jax import lax
from jax.experimental import pallas as pl
from jax.experimental.pallas import tpu as pltpu
from jax.experimental.pallas import tpu_sc as plsc

NC = 2
NS = 16
NW = NC * NS
BG = 128
BG_LOG2 = BG.bit_length() - 1

_DOT_KW = dict(preferred_element_type=jnp.float32, precision=lax.Precision.HIGHEST)


def _dot_t(a, w):
    return lax.dot_general(a, w, (((1,), (1,)), ((), ())), **_DOT_KW)


def _in_fc_body(x_ref, w_ref, b_ref, o_ref):
    o_ref[...] = _dot_t(x_ref[...], w_ref[...]) + b_ref[...]


def _in_fc(x, W, b, rb):
    n, d = x.shape
    return pl.pallas_call(
        _in_fc_body,
        grid=(n // rb,),
        in_specs=[
            pl.BlockSpec((rb, d), lambda i: (i, 0)),
            pl.BlockSpec((d, d), lambda i: (0, 0)),
            pl.BlockSpec((1, d), lambda i: (0, 0)),
        ],
        out_specs=pl.BlockSpec((rb, d), lambda i: (i, 0)),
        out_shape=jax.ShapeDtypeStruct((n, d), jnp.float32),
    )(x, W, b.reshape(1, d))


def _layer_body(p_ref, h_ref, wrel_ref, brel_ref, wroot_ref, o_ref):
    agg = p_ref[0] + p_ref[1]
    acc = _dot_t(agg, wrel_ref[...]) + brel_ref[...]
    acc = acc + _dot_t(h_ref[...], wroot_ref[...])
    o_ref[...] = jnp.maximum(acc, 0.0)


def _layer(parts, h, Wrel, brel, Wroot, rb):
    n, d = h.shape
    return pl.pallas_call(
        _layer_body,
        grid=(n // rb,),
        in_specs=[
            pl.BlockSpec((2, rb, d), lambda i: (0, i, 0)),
            pl.BlockSpec((rb, d), lambda i: (i, 0)),
            pl.BlockSpec((d, d), lambda i: (0, 0)),
            pl.BlockSpec((1, d), lambda i: (0, 0)),
            pl.BlockSpec((d, d), lambda i: (0, 0)),
        ],
        out_specs=pl.BlockSpec((rb, d), lambda i: (i, 0)),
        out_shape=jax.ShapeDtypeStruct((n, d), jnp.float32),
    )(parts, h, Wrel, brel.reshape(1, d), Wroot)


@functools.lru_cache(maxsize=None)
def _make_segsum(n_pad, d, nb, reuse):
    n_sh = n_pad + 8
    rows_t = n_pad // NS
    nck = 8
    k_b = n_pad // 8
    nbh = nb // 2
    assert nbh % nck == 0 and rows_t % 8 == 0 and k_b % 16 == 0
    mesh = plsc.VectorSubcoreMesh(
        core_axis_name="c", subcore_axis_name="s", num_cores=NC, num_subcores=NS
    )

    agg_t = jax.ShapeDtypeStruct((NC, n_pad, d), jnp.float32)
    lists_t = jax.ShapeDtypeStruct((NW * 2, nbh, BG), jnp.int32)
    out_type = agg_t if reuse else (agg_t, lists_t, lists_t)

    @functools.partial(
        pl.kernel,
        out_type=out_type,
        mesh=mesh,
        compiler_params=pltpu.CompilerParams(needs_layout_passes=False),
        scratch_types=[
            pltpu.VMEM((nck, BG), jnp.int32),
            pltpu.VMEM((nck, BG), jnp.int32),
            pltpu.VMEM((k_b,), jnp.int32),
            pltpu.VMEM((nbh, BG), jnp.int32),
            pltpu.VMEM((nbh, BG), jnp.int32),
            pltpu.VMEM((8, d), jnp.float32),
            pltpu.VMEM((BG, d), jnp.float32),
            pltpu.VMEM((BG, d), jnp.float32),
            pltpu.VMEM_SHARED((n_sh, d), jnp.float32),
            pltpu.SemaphoreType.DMA,
            pltpu.SemaphoreType.DMA,
        ],
    )
    def segsum(*args):
        if reuse:
            (h_hbm, ss_hbm, ds_hbm, out_hbm,
             sv, dv, offs, ss2, ds2, zb, rb0, rb1, acc, sem0, sem1) = args
            src_hbm = dst_hbm = None
        else:
            (h_hbm, src_hbm, dst_hbm, out_hbm, ss_hbm, ds_hbm,
             sv, dv, offs, ss2, ds2, zb, rb0, rb1, acc, sem0, sem1) = args
        c = lax.axis_index("c")
        s = lax.axis_index("s")
        wid = s * NC + c

        def zrow(r, carry):
            for cg in range(d // 16):
                zb[r, pl.ds(cg * 16, 16)] = jnp.zeros((16,), jnp.float32)
            return carry
        lax.fori_loop(0, 8, zrow, 0)
        base = s * rows_t

        def zcopy(k, carry):
            off = pl.multiple_of(base + k * 8, 8)
            pltpu.sync_copy(zb, acc.at[pl.ds(off, 8)])
            return carry
        lax.fori_loop(0, rows_t // 8, zcopy, 0)
        plsc.subcore_barrier()

        for ph in range(2):
            hbase = ph * nbh

            if reuse:
                pltpu.async_copy(ss_hbm.at[wid * 2 + ph], ss2, sem0)
                pltpu.async_copy(ds_hbm.at[wid * 2 + ph], ds2, sem1)
                pltpu.make_async_copy(ss_hbm.at[wid * 2 + ph], ss2, sem0).wait()
                pltpu.make_async_copy(ds_hbm.at[wid * 2 + ph], ds2, sem1).wait()
                _sort = False
            else:
                _sort = True

            def hzero(k, carry):
                off = pl.multiple_of(k * 16, 16)
                offs[pl.ds(off, 16)] = jnp.zeros((16,), jnp.int32)
                return carry
            if _sort:
                lax.fori_loop(0, k_b // 16, hzero, 0)

            def hist_chunk(ch, carry):
                off = pl.multiple_of(hbase + ch * nck, nck)
                pltpu.sync_copy(src_hbm.at[wid, pl.ds(off, nck)], sv)
                for r in range(nck):
                    for cg in range(BG // 16):
                        b = sv[r, pl.ds(cg * 16, 16)] >> 3
                        g = plsc.load_gather(offs, [b])
                        cnt, last = plsc.scan_count(b)
                        plsc.store_scatter(offs, [b], g + cnt, mask=last)
                return carry
            if _sort:
                lax.fori_loop(0, nbh // nck, hist_chunk, 0)

            def pfx(k, carry):
                off = pl.multiple_of(k * 16, 16)
                v = offs[pl.ds(off, 16)]
                total = jnp.sum(v)
                offs[pl.ds(off, 16)] = carry + plsc.cumsum(v) - v
                return carry + total
            if _sort:
                lax.fori_loop(0, k_b // 16, pfx, jnp.int32(0))

            def perm_chunk(ch, carry):
                off = pl.multiple_of(hbase + ch * nck, nck)
                pltpu.sync_copy(src_hbm.at[wid, pl.ds(off, nck)], sv)
                pltpu.sync_copy(dst_hbm.at[wid, pl.ds(off, nck)], dv)
                for r in range(nck):
                    for cg in range(BG // 16):
                        sl = pl.ds(cg * 16, 16)
                        s_v = sv[r, sl]
                        d_v = dv[r, sl]
                        b = s_v >> 3
                        g = plsc.load_gather(offs, [b])
                        cnt, last = plsc.scan_count(b)
                        p = g + cnt - 1
                        row = p >> BG_LOG2
                        col = p & (BG - 1)
                        plsc.store_scatter(ss2, [row, col], s_v)
                        plsc.store_scatter(ds2, [row, col], d_v)
                        plsc.store_scatter(offs, [b], p + 1, mask=last)
                return carry
            if _sort:
                lax.fori_loop(0, nbh // nck, perm_chunk, 0)
                pltpu.sync_copy(ss2, ss_hbm.at[wid * 2 + ph])
                pltpu.sync_copy(ds2, ds_hbm.at[wid * 2 + ph])

            pltpu.async_copy(h_hbm.at[ss2.at[0]], rb0, sem0)

            def body(jj, carry2):
                j0 = jj * 2
                j1 = j0 + 1
                pltpu.make_async_copy(h_hbm.at[ss2.at[j0]], rb0, sem0).wait()
                pltpu.async_copy(h_hbm.at[ss2.at[j1]], rb1, sem1)
                pltpu.sync_copy(rb0, acc.at[ds2.at[j0]], add=True)
                pltpu.make_async_copy(h_hbm.at[ss2.at[j1]], rb1, sem1).wait()

                @pl.when(j0 + 2 < nbh)
                def _():
                    pltpu.async_copy(h_hbm.at[ss2.at[j0 + 2]], rb0, sem0)

                pltpu.sync_copy(rb1, acc.at[ds2.at[j1]], add=True)
                return carry2

            lax.fori_loop(0, nbh // 2, body, 0)

        plsc.subcore_barrier()

        pltpu.sync_copy(acc.at[pl.ds(s * rows_t, rows_t)],
                        out_hbm.at[c, pl.ds(s * rows_t, rows_t)])

    return segsum


def kernel(x, edge_index, W_in, b_in, W_rel1, b_rel1, W_root1,
           W_rel2, b_rel2, W_root2):
    n, d = x.shape
    e = edge_index.shape[1]

    per_w = NW * BG
    nb = -(-(-(-e // per_w)) // 16) * 16
    e_pad = nb * per_w

    n_pad = -(-n // (8 * NS)) * 8 * NS

    src = jnp.pad(edge_index[0], (0, e_pad - e))
    dst = jnp.pad(edge_index[1], (0, e_pad - e), constant_values=n_pad)
    src3 = src.reshape(NW, nb, BG)
    dst3 = dst.reshape(NW, nb, BG)

    segsum_sort = _make_segsum(n_pad, d, nb, False)
    segsum_reuse = _make_segsum(n_pad, d, nb, True)
    rb = 1000 if n % 1000 == 0 else 8

    h = _in_fc(x, W_in, b_in, rb)
    parts, ss_lists, ds_lists = segsum_sort(h, src3, dst3)
    h1 = _layer(parts, h, W_rel1, b_rel1, W_root1, rb)
    parts2 = segsum_reuse(h1, ss_lists, ds_lists)
    h2 = _layer(parts2, h1, W_rel2, b_rel2, W_root2, rb)
    return h2
